# baseline (speedup 1.0000x reference)
import functools

import jax
import jax.numpy as jnp
from jax.experimental import pallas as pl
from jax.experimental.pallas import tpu as pltpu

_BF = jnp.bfloat16
_F32 = jnp.float32


def _full_spec(shape):
    nd = len(shape)
    return pl.BlockSpec(shape, lambda i, _nd=nd: (0,) * _nd)


def _taps3():
    return [(di, dj) for di in range(3) for dj in range(3)]


def _stem_pool_body(p_ref, w_ref, s_ref, b_ref, o_ref):
    y = jnp.dot(p_ref[0], w_ref[...], preferred_element_type=_F32)
    y = y * s_ref[...] + b_ref[...]
    y = jnp.maximum(y, 0.0).astype(_BF)
    yr = y.reshape(112, 2, 56, 64)
    ev, od = yr[:, 0], yr[:, 1]
    aw = jnp.maximum(ev, od)
    osw = jnp.concatenate([jnp.zeros((112, 1, 64), _BF), od[:, :-1, :]], axis=1)
    wz = jnp.maximum(aw, osw)

    hh = wz.reshape(56, 2, 56, 64)
    a = jnp.maximum(hh[:, 0], hh[:, 1])
    o = hh[:, 1]
    os_ = jnp.concatenate([jnp.zeros((1, 56, 64), _BF), o[:-1]], axis=0)
    res = jnp.maximum(a, os_).reshape(1, 56, 56, 64)

    o_ref[...] = jnp.zeros_like(o_ref)
    o_ref[:, 1:57, 1:57, :] = res


def _stem_pool(patches, w, s, b):
    bsz = patches.shape[0]
    return pl.pallas_call(
        _stem_pool_body,
        out_shape=jax.ShapeDtypeStruct((bsz, 58, 64, 64), _BF),
        grid=(bsz,),
        in_specs=[
            pl.BlockSpec((1,) + patches.shape[1:], lambda i: (i, 0, 0)),
            _full_spec(w.shape), _full_spec(s.shape), _full_spec(b.shape),
        ],
        out_specs=pl.BlockSpec((1, 58, 64, 64), lambda i: (i, 0, 0, 0)),
        compiler_params=pltpu.CompilerParams(dimension_semantics=("parallel",)),
    )(patches, w, s, b)


def _stem_patches(x):
    bsz = x.shape[0]
    xh = jnp.transpose(x, (0, 2, 3, 1)).astype(_BF)
    xp = jnp.pad(xh, ((0, 0), (3, 3), (3, 3), (0, 0)))
    cols = [xp[:, di:di + 224:2, dj:dj + 224:2, :]
            for di in range(7) for dj in range(7)]
    pat = jnp.stack(cols, axis=3)
    pat = jnp.concatenate([pat[:, :, 0::2], pat[:, :, 1::2]], axis=2)
    return pat.reshape(bsz, 112 * 112, 7 * 7 * 3)


def _xws1_body(x_ref, w_ref, s_ref, b_ref, m_ref, *rest, s, kc, offs, n, res):
    r_ref = rest[0] if res else None
    o_ref = rest[1] if res else rest[0]
    acc = jnp.dot(x_ref[0, offs[0]:offs[0] + s, :], w_ref[0:kc, :],
                  preferred_element_type=_F32)
    acc = acc + jnp.dot(x_ref[0, offs[1]:offs[1] + s, :], w_ref[kc:2 * kc, :],
                        preferred_element_type=_F32)
    acc = acc + jnp.dot(x_ref[0, offs[2]:offs[2] + s, :], w_ref[2 * kc:3 * kc, :],
                        preferred_element_type=_F32)
    y = acc * s_ref[...] + b_ref[...]
    if res:
        y = y + r_ref[0].astype(_F32)
    y = jnp.maximum(y, 0.0) * m_ref[...]
    o_ref[0] = y.astype(_BF)


def _xws1(xw, w, s, b, mask, r=None):
    bsz, t, kc3 = xw.shape
    kc = kc3
    sflat = mask.shape[0]
    wp = (t - sflat) // 2 - 8
    m0 = wp + 8
    offs = (m0 - wp, m0, m0 + wp)
    n = w.shape[1]
    ins = [xw, w, s, b, mask]
    specs = [pl.BlockSpec((1, t, kc3), lambda i: (i, 0, 0)),
             _full_spec(w.shape), _full_spec(s.shape), _full_spec(b.shape),
             _full_spec(mask.shape)]
    if r is not None:
        ins.append(r)
        specs.append(pl.BlockSpec((1, sflat, n), lambda i: (i, 0, 0)))
    body = functools.partial(_xws1_body, s=sflat, kc=kc, offs=offs, n=n,
                             res=r is not None)
    return pl.pallas_call(
        body,
        out_shape=jax.ShapeDtypeStruct((bsz, sflat, n), _BF),
        grid=(bsz,),
        in_specs=specs,
        out_specs=pl.BlockSpec((1, sflat, n), lambda i: (i, 0, 0)),
        compiler_params=pltpu.CompilerParams(dimension_semantics=("parallel",)),
    )(*ins)


def _xws2_body(x0_ref, x1_ref, w_ref, s_ref, b_ref, dw_ref, dss_ref, dsb_ref,
               m_ref, o1_ref, o2_ref, *, s, kc, m0, owp, n):
    acc = jnp.dot(x0_ref[0, 8:8 + s, :], w_ref[0:kc, :],
                  preferred_element_type=_F32)
    acc = acc + jnp.dot(x1_ref[0, 8:8 + s, :], w_ref[kc:2 * kc, :],
                        preferred_element_type=_F32)
    acc = acc + jnp.dot(x0_ref[0, m0:m0 + s, :], w_ref[2 * kc:3 * kc, :],
                        preferred_element_type=_F32)
    y = acc * s_ref[...] + b_ref[...]
    y = jnp.maximum(y, 0.0) * m_ref[...]
    o1_ref[0] = y.astype(_BF)

    c = kc // 3
    dsoff = m0 - owp
    a = x1_ref[0, dsoff:dsoff + s, c:2 * c]
    idn = jnp.dot(a, dw_ref[...], preferred_element_type=_F32)
    idn = (idn * dss_ref[...] + dsb_ref[...]) * m_ref[...]
    o2_ref[0] = idn.astype(_BF)


def _xws2(xw0, xw1, w, s, b, dw, dss, dsb, mask, owp):
    bsz, t, kc3 = xw0.shape
    kc = kc3
    sflat = mask.shape[0]
    m0 = owp + 8
    n = w.shape[1]
    body = functools.partial(_xws2_body, s=sflat, kc=kc, m0=m0, owp=owp, n=n)
    xspec = pl.BlockSpec((1, t, kc3), lambda i: (i, 0, 0))
    return pl.pallas_call(
        body,
        out_shape=(jax.ShapeDtypeStruct((bsz, sflat, n), _BF),
                   jax.ShapeDtypeStruct((bsz, sflat, n), _BF)),
        grid=(bsz,),
        in_specs=[xspec, xspec, _full_spec(w.shape), _full_spec(s.shape),
                  _full_spec(b.shape), _full_spec(dw.shape),
                  _full_spec(dss.shape), _full_spec(dsb.shape),
                  _full_spec(mask.shape)],
        out_specs=(pl.BlockSpec((1, sflat, n), lambda i: (i, 0, 0)),
                   pl.BlockSpec((1, sflat, n), lambda i: (i, 0, 0))),
        compiler_params=pltpu.CompilerParams(dimension_semantics=("parallel",)),
    )(xw0, xw1, w, s, b, dw, dss, dsb, mask)


def _pack_s1(act, wp):
    bsz, sflat, c = act.shape
    z1 = jnp.zeros((bsz, 1, c), _BF)
    xm = jnp.concatenate([z1, act[:, :-1, :]], axis=1)
    xp = jnp.concatenate([act[:, 1:, :], z1], axis=1)
    core = jnp.concatenate([xm, act, xp], axis=2)
    m0 = wp + 8
    return jnp.pad(core, ((0, 0), (m0, m0), (0, 0)))


def _pack_s2(act, hp, wpad, owp, ohp):
    bsz, sflat, c = act.shape
    x4 = act.reshape(bsz, hp, wpad, c)
    x4p = jnp.pad(x4, ((0, 0), (0, 2), (2, 0), (0, 0)))
    outs = []
    for a in range(2):
        gm = x4p[:, a:a + 2 * ohp:2, 0:2 * owp:2, :]
        g0 = x4p[:, a:a + 2 * ohp:2, 1:1 + 2 * owp:2, :]
        gp = x4p[:, a:a + 2 * ohp:2, 2:2 + 2 * owp:2, :]
        xw = jnp.concatenate([gm, g0, gp], axis=3).reshape(
            bsz, ohp * owp, 3 * c)
        m0 = owp + 8
        outs.append(jnp.pad(xw, ((0, 0), (m0, m0), (0, 0))))
    return outs[0], outs[1]


def _flat_mask(hp, wpad, ho, wo):
    q = jnp.arange(hp * wpad)
    r, c = q // wpad, q % wpad
    m = ((r >= 1) & (r <= ho) & (c >= 1) & (c <= wo)).astype(_F32)
    return m.reshape(hp * wpad, 1)


def _conv3s1_body(*refs, bc, ho, wo, cin, n, res, head):
    x_ref, w_ref, s_ref, b_ref = refs[:4]
    idx = 4
    r_ref = None
    if res is not None:
        r_ref = refs[idx]
        idx += 1
    if head:
        fcw_ref, fcb_ref = refs[idx], refs[idx + 1]
        idx += 2
    o_ref = refs[idx]

    m = bc * ho * wo
    acc = jnp.zeros((m, n), _F32)
    for t, (di, dj) in enumerate(_taps3()):
        a = x_ref[:, di:di + ho, dj:dj + wo, :].reshape(m, cin)
        acc = acc + jnp.dot(a, w_ref[t * cin:(t + 1) * cin, :],
                            preferred_element_type=_F32)

    y = acc * s_ref[...] + b_ref[...]
    if res == "padded":
        y = y + r_ref[:, 1:1 + ho, 1:1 + wo, :].reshape(m, n).astype(_F32)
    elif res == "flat":
        y = y + r_ref[...].reshape(m, n).astype(_F32)
    y = jnp.maximum(y, 0.0).astype(_BF)

    if head:
        feat = y.astype(_F32).reshape(bc, ho * wo, n).sum(axis=1) * (1.0 / (ho * wo))
        o_ref[...] = (jnp.dot(feat.astype(_BF), fcw_ref[...],
                              preferred_element_type=_F32) + fcb_ref[...])
    else:
        o_ref[...] = jnp.zeros_like(o_ref)
        o_ref[:, 1:1 + ho, 1:1 + wo, :] = y.reshape(bc, ho, wo, n)


def _conv3s1(xp, w, s, b, *, bc, res=None, r=None, head=False, fcw=None,
             fcb=None):
    bsz, hp, wpd, cin = xp.shape
    ho, wo = hp - 2, wpd - 2
    n = w.shape[1]
    ins = [xp, w, s, b]
    specs = [
        pl.BlockSpec((bc, hp, wpd, cin), lambda i: (i, 0, 0, 0)),
        _full_spec(w.shape), _full_spec(s.shape), _full_spec(b.shape),
    ]
    if res == "padded":
        ins.append(r)
        specs.append(pl.BlockSpec((bc, hp, wpd, n), lambda i: (i, 0, 0, 0)))
    elif res == "flat":
        ins.append(r)
        specs.append(pl.BlockSpec((bc, ho, wo, n), lambda i: (i, 0, 0, 0)))
    if head:
        ins += [fcw, fcb]
        specs += [_full_spec(fcw.shape), _full_spec(fcb.shape)]
        out_shape = jax.ShapeDtypeStruct((bsz, fcw.shape[1]), _F32)
        out_spec = pl.BlockSpec((bc, fcw.shape[1]), lambda i: (i, 0))
    else:
        out_shape = jax.ShapeDtypeStruct((bsz, hp, wpd, n), _BF)
        out_spec = pl.BlockSpec((bc, hp, wpd, n), lambda i: (i, 0, 0, 0))
    body = functools.partial(_conv3s1_body, bc=bc, ho=ho, wo=wo, cin=cin, n=n,
                             res=res, head=head)
    return pl.pallas_call(
        body,
        out_shape=out_shape,
        grid=(bsz // bc,),
        in_specs=specs,
        out_specs=out_spec,
        compiler_params=pltpu.CompilerParams(dimension_semantics=("parallel",)),
    )(*ins)


def _conv3s2_ds_body(p00, p01, p10, p11, w_ref, s_ref, b_ref,
                     dw_ref, dss_ref, dsb_ref, o1_ref, o2_ref,
                     *, bc, ho, wo, cin, n):
    phases = (p00, p01, p10, p11)
    m = bc * ho * wo

    def tap(di, dj):
        p = phases[(di % 2) * 2 + (dj % 2)]
        oi, oj = di // 2, dj // 2
        return p[:, oi:oi + ho, oj:oj + wo, :].reshape(m, cin)

    acc = jnp.zeros((m, n), _F32)
    for t, (di, dj) in enumerate(_taps3()):
        acc = acc + jnp.dot(tap(di, dj), w_ref[t * cin:(t + 1) * cin, :],
                            preferred_element_type=_F32)
    y = jnp.maximum(acc * s_ref[...] + b_ref[...], 0.0).astype(_BF)
    o1_ref[...] = jnp.zeros_like(o1_ref)
    o1_ref[:, 1:1 + ho, 1:1 + wo, :] = y.reshape(bc, ho, wo, n)

    a = p11[:, 0:ho, 0:wo, :].reshape(m, cin)
    idn = jnp.dot(a, dw_ref[...], preferred_element_type=_F32)
    idn = idn * dss_ref[...] + dsb_ref[...]
    o2_ref[...] = idn.astype(_BF).reshape(bc, ho, wo, dw_ref.shape[1])


def _conv3s2_ds(phases, w, s, b, dw, dss, dsb, *, bc):
    bsz, hp, wpd, cin = phases[0].shape
    ho, wo = hp - 1, wpd - 1
    n = w.shape[1]
    pspec = pl.BlockSpec((bc, hp, wpd, cin), lambda i: (i, 0, 0, 0))
    body = functools.partial(_conv3s2_ds_body, bc=bc, ho=ho, wo=wo, cin=cin,
                             n=n)
    return pl.pallas_call(
        body,
        out_shape=(
            jax.ShapeDtypeStruct((bsz, ho + 2, wo + 2, n), _BF),
            jax.ShapeDtypeStruct((bsz, ho, wo, n), _BF),
        ),
        grid=(bsz // bc,),
        in_specs=[pspec, pspec, pspec, pspec,
                  _full_spec(w.shape), _full_spec(s.shape), _full_spec(b.shape),
                  _full_spec(dw.shape), _full_spec(dss.shape),
                  _full_spec(dsb.shape)],
        out_specs=(
            pl.BlockSpec((bc, ho + 2, wo + 2, n), lambda i: (i, 0, 0, 0)),
            pl.BlockSpec((bc, ho, wo, n), lambda i: (i, 0, 0, 0)),
        ),
        compiler_params=pltpu.CompilerParams(dimension_semantics=("parallel",)),
    )(*phases, w, s, b, dw, dss, dsb)


def _phase_split(xp):
    return tuple(xp[:, a::2, b::2, :] for a in range(2) for b in range(2))


def _chunk(bsz, want):
    c = min(want, bsz)
    while bsz % c:
        c -= 1
    return c


def kernel(x, conv1_wmat, conv1_scale, conv1_bias, l0b0_c1_wmat, l0b0_c1_scale, l0b0_c1_bias, l0b0_c2_wmat, l0b0_c2_scale, l0b0_c2_bias, l0b1_c1_wmat, l0b1_c1_scale, l0b1_c1_bias, l0b1_c2_wmat, l0b1_c2_scale, l0b1_c2_bias, l1b0_c1_wmat, l1b0_c1_scale, l1b0_c1_bias, l1b0_c2_wmat, l1b0_c2_scale, l1b0_c2_bias, l1b0_ds_wmat, l1b0_ds_scale, l1b0_ds_bias, l1b1_c1_wmat, l1b1_c1_scale, l1b1_c1_bias, l1b1_c2_wmat, l1b1_c2_scale, l1b1_c2_bias, l2b0_c1_wmat, l2b0_c1_scale, l2b0_c1_bias, l2b0_c2_wmat, l2b0_c2_scale, l2b0_c2_bias, l2b0_ds_wmat, l2b0_ds_scale, l2b0_ds_bias, l2b1_c1_wmat, l2b1_c1_scale, l2b1_c1_bias, l2b1_c2_wmat, l2b1_c2_scale, l2b1_c2_bias, l3b0_c1_wmat, l3b0_c1_scale, l3b0_c1_bias, l3b0_c2_wmat, l3b0_c2_scale, l3b0_c2_bias, l3b0_ds_wmat, l3b0_ds_scale, l3b0_ds_bias, l3b1_c1_wmat, l3b1_c1_scale, l3b1_c1_bias, l3b1_c2_wmat, l3b1_c2_scale, l3b1_c2_bias, fc_w, fc_b):
    x = x.reshape(-1, 3, 224, 224)
    bsz = x.shape[0]
    bc4 = _chunk(bsz, 32)

    p1 = _stem_pool(_stem_patches(x), conv1_wmat, conv1_scale,
                    conv1_bias).reshape(bsz, 58 * 64, 64)

    mk1 = _flat_mask(58, 64, 56, 56)
    y = _xws1(_pack_s1(p1, 64), l0b0_c1_wmat, l0b0_c1_scale, l0b0_c1_bias, mk1)
    p2 = _xws1(_pack_s1(y, 64), l0b0_c2_wmat, l0b0_c2_scale, l0b0_c2_bias,
               mk1, r=p1)
    y = _xws1(_pack_s1(p2, 64), l0b1_c1_wmat, l0b1_c1_scale, l0b1_c1_bias, mk1)
    p3 = _xws1(_pack_s1(y, 64), l0b1_c2_wmat, l0b1_c2_scale, l0b1_c2_bias,
               mk1, r=p2)

    mk2 = _flat_mask(30, 32, 28, 28)
    xw0, xw1 = _pack_s2(p3, 58, 64, 32, 30)
    y1, idn = _xws2(xw0, xw1, l1b0_c1_wmat, l1b0_c1_scale, l1b0_c1_bias,
                    l1b0_ds_wmat, l1b0_ds_scale, l1b0_ds_bias, mk2, 32)
    p4 = _xws1(_pack_s1(y1, 32), l1b0_c2_wmat, l1b0_c2_scale, l1b0_c2_bias,
               mk2, r=idn)
    y = _xws1(_pack_s1(p4, 32), l1b1_c1_wmat, l1b1_c1_scale, l1b1_c1_bias, mk2)
    p5 = _xws1(_pack_s1(y, 32), l1b1_c2_wmat, l1b1_c2_scale, l1b1_c2_bias,
               mk2, r=p4)

    mk3 = _flat_mask(16, 16, 14, 14)
    xw0, xw1 = _pack_s2(p5, 30, 32, 16, 16)
    y1, idn = _xws2(xw0, xw1, l2b0_c1_wmat, l2b0_c1_scale, l2b0_c1_bias,
                    l2b0_ds_wmat, l2b0_ds_scale, l2b0_ds_bias, mk3, 16)
    p6 = _xws1(_pack_s1(y1, 16), l2b0_c2_wmat, l2b0_c2_scale, l2b0_c2_bias,
               mk3, r=idn)
    y = _xws1(_pack_s1(p6, 16), l2b1_c1_wmat, l2b1_c1_scale, l2b1_c1_bias, mk3)
    p7 = _xws1(_pack_s1(y, 16), l2b1_c2_wmat, l2b1_c2_scale, l2b1_c2_bias,
               mk3, r=p6)

    p7_4d = p7.reshape(bsz, 16, 16, 256)
    y1p, idn = _conv3s2_ds(_phase_split(p7_4d), l3b0_c1_wmat, l3b0_c1_scale,
                           l3b0_c1_bias, l3b0_ds_wmat, l3b0_ds_scale,
                           l3b0_ds_bias, bc=bc4)
    p8 = _conv3s1(y1p, l3b0_c2_wmat, l3b0_c2_scale, l3b0_c2_bias, bc=bc4,
                  res="flat", r=idn)
    y = _conv3s1(p8, l3b1_c1_wmat, l3b1_c1_scale, l3b1_c1_bias, bc=bc4)
    return _conv3s1(y, l3b1_c2_wmat, l3b1_c2_scale, l3b1_c2_bias, bc=bc4,
                    res="padded", r=p8, head=True, fcw=fc_w, fcb=fc_b)


# speedup vs baseline: 1.1377x; 1.1377x over previous
import functools

import jax
import jax.numpy as jnp
from jax.experimental import pallas as pl
from jax.experimental.pallas import tpu as pltpu

_BF = jnp.bfloat16
_F32 = jnp.float32


def _full_spec(shape):
    nd = len(shape)
    return pl.BlockSpec(shape, lambda i, _nd=nd: (0,) * _nd)


def _taps3():
    return [(di, dj) for di in range(3) for dj in range(3)]


def _stem_pool_body(p_ref, w_ref, s_ref, b_ref, o_ref):
    y = jnp.dot(p_ref[0], w_ref[...], preferred_element_type=_F32)
    y = y * s_ref[...] + b_ref[...]
    y = jnp.maximum(y, 0.0).astype(_BF)
    yr = y.reshape(112, 2, 56, 64)
    ev, od = yr[:, 0], yr[:, 1]
    aw = jnp.maximum(ev, od)
    osw = jnp.concatenate([jnp.zeros((112, 1, 64), _BF), od[:, :-1, :]], axis=1)
    wz = jnp.maximum(aw, osw)

    hh = wz.reshape(56, 2, 56, 64)
    a = jnp.maximum(hh[:, 0], hh[:, 1])
    o = hh[:, 1]
    os_ = jnp.concatenate([jnp.zeros((1, 56, 64), _BF), o[:-1]], axis=0)
    res = jnp.maximum(a, os_).reshape(1, 56, 56, 64)

    o_ref[...] = jnp.zeros_like(o_ref)
    o_ref[:, 1:57, 1:57, :] = res


def _stem_pool(patches, w, s, b):
    bsz = patches.shape[0]
    return pl.pallas_call(
        _stem_pool_body,
        out_shape=jax.ShapeDtypeStruct((bsz, 58, 64, 64), _BF),
        grid=(bsz,),
        in_specs=[
            pl.BlockSpec((1,) + patches.shape[1:], lambda i: (i, 0, 0)),
            _full_spec(w.shape), _full_spec(s.shape), _full_spec(b.shape),
        ],
        out_specs=pl.BlockSpec((1, 58, 64, 64), lambda i: (i, 0, 0, 0)),
        compiler_params=pltpu.CompilerParams(dimension_semantics=("parallel",)),
    )(patches, w, s, b)


def _stem_patches(x):
    bsz = x.shape[0]
    xh = jnp.transpose(x, (0, 2, 3, 1)).astype(_BF)
    xp = jnp.pad(xh, ((0, 0), (3, 3), (3, 3), (0, 0)))
    cols = [xp[:, di:di + 224:2, dj:dj + 224:2, :]
            for di in range(7) for dj in range(7)]
    pat = jnp.stack(cols, axis=3)
    pat = jnp.concatenate([pat[:, :, 0::2], pat[:, :, 1::2]], axis=2)
    return pat.reshape(bsz, 112 * 112, 7 * 7 * 3)


def _fs1_body(x_ref, w_ref, s_ref, b_ref, m_ref, *rest, sflat, wp, c, n, res):
    r_ref = rest[0] if res else None
    o_ref = rest[1] if res else rest[0]
    xw_ref = rest[2] if res else rest[1]

    m0 = wp + 8
    t = sflat + 2 * wp + 16
    xv = x_ref[0]
    zc = jnp.zeros((m0 + 2, 3 * c), _BF)
    xw_ref[0:m0 + 2, :] = zc
    xw_ref[t - m0 - 2:t, :] = zc
    xw_ref[m0 + 1:m0 + 1 + sflat, 0:c] = xv
    xw_ref[m0:m0 + sflat, c:2 * c] = xv
    xw_ref[m0 - 1:m0 - 1 + sflat, 2 * c:3 * c] = xv

    kc = 3 * c
    acc = jnp.dot(xw_ref[m0 - wp:m0 - wp + sflat, :], w_ref[0:kc, :],
                  preferred_element_type=_F32)
    acc = acc + jnp.dot(xw_ref[m0:m0 + sflat, :], w_ref[kc:2 * kc, :],
                        preferred_element_type=_F32)
    acc = acc + jnp.dot(xw_ref[m0 + wp:m0 + wp + sflat, :],
                        w_ref[2 * kc:3 * kc, :],
                        preferred_element_type=_F32)
    y = acc * s_ref[...] + b_ref[...]
    if res:
        y = y + r_ref[0].astype(_F32)
    y = jnp.maximum(y, 0.0) * m_ref[...]
    o_ref[0] = y.astype(_BF)


def _fs1(act, w, s, b, mask, wp, r=None):
    bsz, sflat, c = act.shape
    n = w.shape[1]
    ins = [act, w, s, b, mask]
    specs = [pl.BlockSpec((1, sflat, c), lambda i: (i, 0, 0)),
             _full_spec(w.shape), _full_spec(s.shape), _full_spec(b.shape),
             _full_spec(mask.shape)]
    if r is not None:
        ins.append(r)
        specs.append(pl.BlockSpec((1, sflat, n), lambda i: (i, 0, 0)))
    body = functools.partial(_fs1_body, sflat=sflat, wp=wp, c=c, n=n,
                             res=r is not None)
    return pl.pallas_call(
        body,
        out_shape=jax.ShapeDtypeStruct((bsz, sflat, n), _BF),
        grid=(bsz,),
        in_specs=specs,
        out_specs=pl.BlockSpec((1, sflat, n), lambda i: (i, 0, 0)),
        scratch_shapes=[pltpu.VMEM((sflat + 2 * wp + 16, 3 * c), _BF)],
        compiler_params=pltpu.CompilerParams(dimension_semantics=("parallel",)),
    )(*ins)


def _flat_mask(hp, wpad, ho, wo):
    q = jnp.arange(hp * wpad)
    r, c = q // wpad, q % wpad
    m = ((r >= 1) & (r <= ho) & (c >= 1) & (c <= wo)).astype(_F32)
    return m.reshape(hp * wpad, 1)


def _conv3s2_ds_body(p00, p01, p10, p11, w_ref, s_ref, b_ref,
                     dw_ref, dss_ref, dsb_ref, o1_ref, o2_ref,
                     *, bc, ho, wo, cin, n):
    phases = (p00, p01, p10, p11)
    m = bc * ho * wo

    def tap(di, dj):
        p = phases[(di % 2) * 2 + (dj % 2)]
        oi, oj = di // 2, dj // 2
        return p[:, oi:oi + ho, oj:oj + wo, :].reshape(m, cin)

    acc = jnp.zeros((m, n), _F32)
    for t, (di, dj) in enumerate(_taps3()):
        acc = acc + jnp.dot(tap(di, dj), w_ref[t * cin:(t + 1) * cin, :],
                            preferred_element_type=_F32)
    y = jnp.maximum(acc * s_ref[...] + b_ref[...], 0.0).astype(_BF)
    o1_ref[...] = jnp.zeros_like(o1_ref)
    o1_ref[:, 1:1 + ho, 1:1 + wo, :] = y.reshape(bc, ho, wo, n)

    a = p11[:, 0:ho, 0:wo, :].reshape(m, cin)
    idn = jnp.dot(a, dw_ref[...], preferred_element_type=_F32)
    idn = (idn * dss_ref[...] + dsb_ref[...]).astype(_BF)
    o2_ref[...] = jnp.zeros_like(o2_ref)
    o2_ref[:, 1:1 + ho, 1:1 + wo, :] = idn.reshape(bc, ho, wo, n)


def _conv3s2_ds(phases, w, s, b, dw, dss, dsb, *, bc, ho, wo, wpad):
    bsz, hp, wpp, cin = phases[0].shape
    n = w.shape[1]
    pspec = pl.BlockSpec((bc, hp, wpp, cin), lambda i: (i, 0, 0, 0))
    body = functools.partial(_conv3s2_ds_body, bc=bc, ho=ho, wo=wo, cin=cin,
                             n=n)
    oshape = jax.ShapeDtypeStruct((bsz, ho + 2, wpad, n), _BF)
    ospec = pl.BlockSpec((bc, ho + 2, wpad, n), lambda i: (i, 0, 0, 0))
    return pl.pallas_call(
        body,
        out_shape=(oshape, oshape),
        grid=(bsz // bc,),
        in_specs=[pspec, pspec, pspec, pspec,
                  _full_spec(w.shape), _full_spec(s.shape), _full_spec(b.shape),
                  _full_spec(dw.shape), _full_spec(dss.shape),
                  _full_spec(dsb.shape)],
        out_specs=(ospec, ospec),
        compiler_params=pltpu.CompilerParams(dimension_semantics=("parallel",)),
    )(*phases, w, s, b, dw, dss, dsb)


def _conv3s1_body(*refs, bc, ho, wo, cin, n, res, head):
    x_ref, w_ref, s_ref, b_ref = refs[:4]
    idx = 4
    r_ref = None
    if res is not None:
        r_ref = refs[idx]
        idx += 1
    if head:
        fcw_ref, fcb_ref = refs[idx], refs[idx + 1]
        idx += 2
    o_ref = refs[idx]

    m = bc * ho * wo
    acc = jnp.zeros((m, n), _F32)
    for t, (di, dj) in enumerate(_taps3()):
        a = x_ref[:, di:di + ho, dj:dj + wo, :].reshape(m, cin)
        acc = acc + jnp.dot(a, w_ref[t * cin:(t + 1) * cin, :],
                            preferred_element_type=_F32)

    y = acc * s_ref[...] + b_ref[...]
    if res == "padded":
        y = y + r_ref[:, 1:1 + ho, 1:1 + wo, :].reshape(m, n).astype(_F32)
    elif res == "flat":
        y = y + r_ref[...].reshape(m, n).astype(_F32)
    y = jnp.maximum(y, 0.0).astype(_BF)

    if head:
        feat = y.astype(_F32).reshape(bc, ho * wo, n).sum(axis=1) * (1.0 / (ho * wo))
        o_ref[...] = (jnp.dot(feat.astype(_BF), fcw_ref[...],
                              preferred_element_type=_F32) + fcb_ref[...])
    else:
        o_ref[...] = jnp.zeros_like(o_ref)
        o_ref[:, 1:1 + ho, 1:1 + wo, :] = y.reshape(bc, ho, wo, n)


def _conv3s1(xp, w, s, b, *, bc, res=None, r=None, head=False, fcw=None,
             fcb=None):
    bsz, hp, wpd, cin = xp.shape
    ho, wo = hp - 2, wpd - 2
    n = w.shape[1]
    ins = [xp, w, s, b]
    specs = [
        pl.BlockSpec((bc, hp, wpd, cin), lambda i: (i, 0, 0, 0)),
        _full_spec(w.shape), _full_spec(s.shape), _full_spec(b.shape),
    ]
    if res == "padded":
        ins.append(r)
        specs.append(pl.BlockSpec((bc, hp, wpd, n), lambda i: (i, 0, 0, 0)))
    elif res == "flat":
        ins.append(r)
        specs.append(pl.BlockSpec((bc, ho, wo, n), lambda i: (i, 0, 0, 0)))
    if head:
        ins += [fcw, fcb]
        specs += [_full_spec(fcw.shape), _full_spec(fcb.shape)]
        out_shape = jax.ShapeDtypeStruct((bsz, fcw.shape[1]), _F32)
        out_spec = pl.BlockSpec((bc, fcw.shape[1]), lambda i: (i, 0))
    else:
        out_shape = jax.ShapeDtypeStruct((bsz, hp, wpd, n), _BF)
        out_spec = pl.BlockSpec((bc, hp, wpd, n), lambda i: (i, 0, 0, 0))
    body = functools.partial(_conv3s1_body, bc=bc, ho=ho, wo=wo, cin=cin, n=n,
                             res=res, head=head)
    return pl.pallas_call(
        body,
        out_shape=out_shape,
        grid=(bsz // bc,),
        in_specs=specs,
        out_specs=out_spec,
        compiler_params=pltpu.CompilerParams(dimension_semantics=("parallel",)),
    )(*ins)


def _phase_split(xp):
    return tuple(xp[:, a::2, b::2, :] for a in range(2) for b in range(2))


def _chunk(bsz, want):
    c = min(want, bsz)
    while bsz % c:
        c -= 1
    return c


def kernel(x, conv1_wmat, conv1_scale, conv1_bias, l0b0_c1_wmat, l0b0_c1_scale, l0b0_c1_bias, l0b0_c2_wmat, l0b0_c2_scale, l0b0_c2_bias, l0b1_c1_wmat, l0b1_c1_scale, l0b1_c1_bias, l0b1_c2_wmat, l0b1_c2_scale, l0b1_c2_bias, l1b0_c1_wmat, l1b0_c1_scale, l1b0_c1_bias, l1b0_c2_wmat, l1b0_c2_scale, l1b0_c2_bias, l1b0_ds_wmat, l1b0_ds_scale, l1b0_ds_bias, l1b1_c1_wmat, l1b1_c1_scale, l1b1_c1_bias, l1b1_c2_wmat, l1b1_c2_scale, l1b1_c2_bias, l2b0_c1_wmat, l2b0_c1_scale, l2b0_c1_bias, l2b0_c2_wmat, l2b0_c2_scale, l2b0_c2_bias, l2b0_ds_wmat, l2b0_ds_scale, l2b0_ds_bias, l2b1_c1_wmat, l2b1_c1_scale, l2b1_c1_bias, l2b1_c2_wmat, l2b1_c2_scale, l2b1_c2_bias, l3b0_c1_wmat, l3b0_c1_scale, l3b0_c1_bias, l3b0_c2_wmat, l3b0_c2_scale, l3b0_c2_bias, l3b0_ds_wmat, l3b0_ds_scale, l3b0_ds_bias, l3b1_c1_wmat, l3b1_c1_scale, l3b1_c1_bias, l3b1_c2_wmat, l3b1_c2_scale, l3b1_c2_bias, fc_w, fc_b):
    x = x.reshape(-1, 3, 224, 224)
    bsz = x.shape[0]
    bc2 = _chunk(bsz, 4)
    bc3 = _chunk(bsz, 16)
    bc4 = _chunk(bsz, 32)

    p1 = _stem_pool(_stem_patches(x), conv1_wmat, conv1_scale,
                    conv1_bias).reshape(bsz, 58 * 64, 64)

    mk1 = _flat_mask(58, 64, 56, 56)
    y = _fs1(p1, l0b0_c1_wmat, l0b0_c1_scale, l0b0_c1_bias, mk1, 64)
    p2 = _fs1(y, l0b0_c2_wmat, l0b0_c2_scale, l0b0_c2_bias, mk1, 64, r=p1)
    y = _fs1(p2, l0b1_c1_wmat, l0b1_c1_scale, l0b1_c1_bias, mk1, 64)
    p3 = _fs1(y, l0b1_c2_wmat, l0b1_c2_scale, l0b1_c2_bias, mk1, 64, r=p2)

    mk2 = _flat_mask(30, 32, 28, 28)
    y1, idn = _conv3s2_ds(_phase_split(p3.reshape(bsz, 58, 64, 64)),
                          l1b0_c1_wmat, l1b0_c1_scale, l1b0_c1_bias,
                          l1b0_ds_wmat, l1b0_ds_scale, l1b0_ds_bias,
                          bc=bc2, ho=28, wo=28, wpad=32)
    y1 = y1.reshape(bsz, 960, 128)
    idn = idn.reshape(bsz, 960, 128)
    p4 = _fs1(y1, l1b0_c2_wmat, l1b0_c2_scale, l1b0_c2_bias, mk2, 32, r=idn)
    y = _fs1(p4, l1b1_c1_wmat, l1b1_c1_scale, l1b1_c1_bias, mk2, 32)
    p5 = _fs1(y, l1b1_c2_wmat, l1b1_c2_scale, l1b1_c2_bias, mk2, 32, r=p4)

    mk3 = _flat_mask(16, 16, 14, 14)
    y1, idn = _conv3s2_ds(_phase_split(p5.reshape(bsz, 30, 32, 128)),
                          l2b0_c1_wmat, l2b0_c1_scale, l2b0_c1_bias,
                          l2b0_ds_wmat, l2b0_ds_scale, l2b0_ds_bias,
                          bc=bc3, ho=14, wo=14, wpad=16)
    y1 = y1.reshape(bsz, 256, 256)
    idn = idn.reshape(bsz, 256, 256)
    p6 = _fs1(y1, l2b0_c2_wmat, l2b0_c2_scale, l2b0_c2_bias, mk3, 16, r=idn)
    y = _fs1(p6, l2b1_c1_wmat, l2b1_c1_scale, l2b1_c1_bias, mk3, 16)
    p7 = _fs1(y, l2b1_c2_wmat, l2b1_c2_scale, l2b1_c2_bias, mk3, 16, r=p6)

    y1p, idn = _conv3s2_ds(_phase_split(p7.reshape(bsz, 16, 16, 256)),
                           l3b0_c1_wmat, l3b0_c1_scale, l3b0_c1_bias,
                           l3b0_ds_wmat, l3b0_ds_scale, l3b0_ds_bias,
                           bc=bc4, ho=7, wo=7, wpad=9)
    p8 = _conv3s1(y1p, l3b0_c2_wmat, l3b0_c2_scale, l3b0_c2_bias, bc=bc4,
                  res="padded", r=idn)
    y = _conv3s1(p8, l3b1_c1_wmat, l3b1_c1_scale, l3b1_c1_bias, bc=bc4)
    return _conv3s1(y, l3b1_c2_wmat, l3b1_c2_scale, l3b1_c2_bias, bc=bc4,
                    res="padded", r=p8, head=True, fcw=fc_w, fcb=fc_b)


# speedup vs baseline: 2.4272x; 2.1335x over previous
import functools

import jax
import jax.numpy as jnp
from jax.experimental import pallas as pl
from jax.experimental.pallas import tpu as pltpu

_BF = jnp.bfloat16
_F32 = jnp.float32


def _full_spec(shape):
    nd = len(shape)
    return pl.BlockSpec(shape, lambda i, _nd=nd: (0,) * _nd)


def _taps3():
    return [(di, dj) for di in range(3) for dj in range(3)]


_SS = 115 * 120
_SM0 = 248
_ST = _SS + _SM0 + 128


def _stem_pool_body(x_ref, w_ref, s_ref, b_ref, m_ref, o_ref, xw_ref):
    xv = x_ref[0]
    zc = jnp.zeros((_SM0 + 2, 64), _BF)
    xw_ref[0:_SM0 + 2, :] = zc
    xw_ref[_ST - 130:_ST, :] = jnp.zeros((130, 64), _BF)
    for j in range(4):
        xw_ref[_SM0 + 2 - j:_SM0 + 2 - j + _SS, 16 * j:16 * j + 16] = xv
    acc = jnp.dot(xw_ref[_SM0 - 240:_SM0 - 240 + _SS, :], w_ref[0:64, :],
                  preferred_element_type=_F32)
    for jr in range(1, 4):
        off = _SM0 + (jr - 2) * 120
        acc = acc + jnp.dot(xw_ref[off:off + _SS, :],
                            w_ref[64 * jr:64 * jr + 64, :],
                            preferred_element_type=_F32)
    y = acc * s_ref[...] + b_ref[...]
    y = jnp.maximum(y, 0.0) * m_ref[...]
    z = y[240:240 + 112 * 120].astype(_BF).reshape(112, 120, 64)

    hh = z.reshape(56, 2, 120, 64)
    a = jnp.maximum(hh[:, 0], hh[:, 1])
    o = hh[:, 1]
    os_ = jnp.concatenate([jnp.zeros((1, 120, 64), _BF), o[:-1]], axis=0)
    hz = jnp.maximum(a, os_).reshape(6720, 64)
    zm = jnp.concatenate([jnp.zeros((1, 64), _BF), hz[:-1]], axis=0)
    zp = jnp.concatenate([hz[1:], jnp.zeros((1, 64), _BF)], axis=0)
    o_ref[0] = jnp.maximum(jnp.maximum(hz, zm), zp)


def _stem_pool(xs, w, s, b, mask):
    bsz = xs.shape[0]
    return pl.pallas_call(
        _stem_pool_body,
        out_shape=jax.ShapeDtypeStruct((bsz, 6720, 64), _BF),
        grid=(bsz,),
        in_specs=[
            pl.BlockSpec((1, _SS, 16), lambda i: (i, 0, 0)),
            _full_spec(w.shape), _full_spec(s.shape), _full_spec(b.shape),
            _full_spec(mask.shape),
        ],
        out_specs=pl.BlockSpec((1, 6720, 64), lambda i: (i, 0, 0)),
        scratch_shapes=[pltpu.VMEM((_ST, 64), _BF)],
        compiler_params=pltpu.CompilerParams(dimension_semantics=("parallel",)),
    )(xs, w, s, b, mask)


def _stem_s2d(x):
    bsz = x.shape[0]
    x6 = x.reshape(bsz, 3, 112, 2, 112, 2)
    xs = jnp.transpose(x6, (0, 2, 4, 3, 5, 1)).reshape(
        bsz, 112, 112, 12).astype(_BF)
    xsp = jnp.pad(xs, ((0, 0), (2, 1), (2, 6), (0, 4)))
    return xsp.reshape(bsz, _SS, 16)


def _stem_w16(wmat):
    k = jnp.arange(256)
    jr, rem = k // 64, k % 64
    jc, l = rem // 16, rem % 16
    a, bb, c = l // 6, (l % 6) // 3, l % 3
    kh = 2 * (jr - 2) + a + 3
    kw = 2 * (jc - 2) + bb + 3
    valid = (l < 12) & (kh >= 0) & (kh < 7) & (kw >= 0) & (kw < 7)
    row = jnp.clip(kh * 21 + kw * 3 + c, 0, 146)
    return jnp.where(valid[:, None], wmat[row, :], jnp.zeros((), wmat.dtype))


def _stem_mask():
    q = jnp.arange(_SS)
    r, c = q // 120, q % 120
    m = ((r >= 2) & (r < 114) & (c >= 2) & (c < 114)).astype(_F32)
    return m.reshape(_SS, 1)


def _fs1_body(x_ref, w_ref, s_ref, b_ref, m_ref, *rest, sflat, wp, c, n, res):
    r_ref = rest[0] if res else None
    o_ref = rest[1] if res else rest[0]
    xw_ref = rest[2] if res else rest[1]

    m0 = wp + 8
    t = sflat + 2 * wp + 16
    xv = x_ref[0]
    zc = jnp.zeros((m0 + 2, 3 * c), _BF)
    xw_ref[0:m0 + 2, :] = zc
    xw_ref[t - m0 - 2:t, :] = zc
    xw_ref[m0 + 1:m0 + 1 + sflat, 0:c] = xv
    xw_ref[m0:m0 + sflat, c:2 * c] = xv
    xw_ref[m0 - 1:m0 - 1 + sflat, 2 * c:3 * c] = xv

    kc = 3 * c
    acc = jnp.dot(xw_ref[m0 - wp:m0 - wp + sflat, :], w_ref[0:kc, :],
                  preferred_element_type=_F32)
    acc = acc + jnp.dot(xw_ref[m0:m0 + sflat, :], w_ref[kc:2 * kc, :],
                        preferred_element_type=_F32)
    acc = acc + jnp.dot(xw_ref[m0 + wp:m0 + wp + sflat, :],
                        w_ref[2 * kc:3 * kc, :],
                        preferred_element_type=_F32)
    y = acc * s_ref[...] + b_ref[...]
    if res:
        y = y + r_ref[0].astype(_F32)
    y = jnp.maximum(y, 0.0) * m_ref[...]
    o_ref[0] = y.astype(_BF)


def _fs1(act, w, s, b, mask, wp, r=None):
    bsz, sflat, c = act.shape
    n = w.shape[1]
    ins = [act, w, s, b, mask]
    specs = [pl.BlockSpec((1, sflat, c), lambda i: (i, 0, 0)),
             _full_spec(w.shape), _full_spec(s.shape), _full_spec(b.shape),
             _full_spec(mask.shape)]
    if r is not None:
        ins.append(r)
        specs.append(pl.BlockSpec((1, sflat, n), lambda i: (i, 0, 0)))
    body = functools.partial(_fs1_body, sflat=sflat, wp=wp, c=c, n=n,
                             res=r is not None)
    return pl.pallas_call(
        body,
        out_shape=jax.ShapeDtypeStruct((bsz, sflat, n), _BF),
        grid=(bsz,),
        in_specs=specs,
        out_specs=pl.BlockSpec((1, sflat, n), lambda i: (i, 0, 0)),
        scratch_shapes=[pltpu.VMEM((sflat + 2 * wp + 16, 3 * c), _BF)],
        compiler_params=pltpu.CompilerParams(dimension_semantics=("parallel",)),
    )(*ins)


def _flat_mask(hp, wpad, ho, wo):
    q = jnp.arange(hp * wpad)
    r, c = q // wpad, q % wpad
    m = ((r >= 1) & (r <= ho) & (c >= 1) & (c <= wo)).astype(_F32)
    return m.reshape(hp * wpad, 1)


def _conv3s2_ds_body(p00, p01, p10, p11, w_ref, s_ref, b_ref,
                     dw_ref, dss_ref, dsb_ref, o1_ref, o2_ref,
                     *, bc, ho, wo, cin, n):
    phases = (p00, p01, p10, p11)
    m = bc * ho * wo

    def tap(di, dj):
        p = phases[(di % 2) * 2 + (dj % 2)]
        oi, oj = di // 2, dj // 2
        return p[:, oi:oi + ho, oj:oj + wo, :].reshape(m, cin)

    acc = jnp.zeros((m, n), _F32)
    for t, (di, dj) in enumerate(_taps3()):
        acc = acc + jnp.dot(tap(di, dj), w_ref[t * cin:(t + 1) * cin, :],
                            preferred_element_type=_F32)
    y = jnp.maximum(acc * s_ref[...] + b_ref[...], 0.0).astype(_BF)
    o1_ref[...] = jnp.zeros_like(o1_ref)
    o1_ref[:, 1:1 + ho, 1:1 + wo, :] = y.reshape(bc, ho, wo, n)

    a = p11[:, 0:ho, 0:wo, :].reshape(m, cin)
    idn = jnp.dot(a, dw_ref[...], preferred_element_type=_F32)
    idn = (idn * dss_ref[...] + dsb_ref[...]).astype(_BF)
    o2_ref[...] = jnp.zeros_like(o2_ref)
    o2_ref[:, 1:1 + ho, 1:1 + wo, :] = idn.reshape(bc, ho, wo, n)


def _conv3s2_ds(phases, w, s, b, dw, dss, dsb, *, bc, ho, wo, wpad):
    bsz, hp, wpp, cin = phases[0].shape
    n = w.shape[1]
    pspec = pl.BlockSpec((bc, hp, wpp, cin), lambda i: (i, 0, 0, 0))
    body = functools.partial(_conv3s2_ds_body, bc=bc, ho=ho, wo=wo, cin=cin,
                             n=n)
    oshape = jax.ShapeDtypeStruct((bsz, ho + 2, wpad, n), _BF)
    ospec = pl.BlockSpec((bc, ho + 2, wpad, n), lambda i: (i, 0, 0, 0))
    return pl.pallas_call(
        body,
        out_shape=(oshape, oshape),
        grid=(bsz // bc,),
        in_specs=[pspec, pspec, pspec, pspec,
                  _full_spec(w.shape), _full_spec(s.shape), _full_spec(b.shape),
                  _full_spec(dw.shape), _full_spec(dss.shape),
                  _full_spec(dsb.shape)],
        out_specs=(ospec, ospec),
        compiler_params=pltpu.CompilerParams(dimension_semantics=("parallel",)),
    )(*phases, w, s, b, dw, dss, dsb)


def _conv3s1_body(*refs, bc, ho, wo, cin, n, res, head):
    x_ref, w_ref, s_ref, b_ref = refs[:4]
    idx = 4
    r_ref = None
    if res is not None:
        r_ref = refs[idx]
        idx += 1
    if head:
        fcw_ref, fcb_ref = refs[idx], refs[idx + 1]
        idx += 2
    o_ref = refs[idx]

    m = bc * ho * wo
    acc = jnp.zeros((m, n), _F32)
    for t, (di, dj) in enumerate(_taps3()):
        a = x_ref[:, di:di + ho, dj:dj + wo, :].reshape(m, cin)
        acc = acc + jnp.dot(a, w_ref[t * cin:(t + 1) * cin, :],
                            preferred_element_type=_F32)

    y = acc * s_ref[...] + b_ref[...]
    if res == "padded":
        y = y + r_ref[:, 1:1 + ho, 1:1 + wo, :].reshape(m, n).astype(_F32)
    elif res == "flat":
        y = y + r_ref[...].reshape(m, n).astype(_F32)
    y = jnp.maximum(y, 0.0).astype(_BF)

    if head:
        feat = y.astype(_F32).reshape(bc, ho * wo, n).sum(axis=1) * (1.0 / (ho * wo))
        o_ref[...] = (jnp.dot(feat.astype(_BF), fcw_ref[...],
                              preferred_element_type=_F32) + fcb_ref[...])
    else:
        o_ref[...] = jnp.zeros_like(o_ref)
        o_ref[:, 1:1 + ho, 1:1 + wo, :] = y.reshape(bc, ho, wo, n)


def _conv3s1(xp, w, s, b, *, bc, res=None, r=None, head=False, fcw=None,
             fcb=None):
    bsz, hp, wpd, cin = xp.shape
    ho, wo = hp - 2, wpd - 2
    n = w.shape[1]
    ins = [xp, w, s, b]
    specs = [
        pl.BlockSpec((bc, hp, wpd, cin), lambda i: (i, 0, 0, 0)),
        _full_spec(w.shape), _full_spec(s.shape), _full_spec(b.shape),
    ]
    if res == "padded":
        ins.append(r)
        specs.append(pl.BlockSpec((bc, hp, wpd, n), lambda i: (i, 0, 0, 0)))
    elif res == "flat":
        ins.append(r)
        specs.append(pl.BlockSpec((bc, ho, wo, n), lambda i: (i, 0, 0, 0)))
    if head:
        ins += [fcw, fcb]
        specs += [_full_spec(fcw.shape), _full_spec(fcb.shape)]
        out_shape = jax.ShapeDtypeStruct((bsz, fcw.shape[1]), _F32)
        out_spec = pl.BlockSpec((bc, fcw.shape[1]), lambda i: (i, 0))
    else:
        out_shape = jax.ShapeDtypeStruct((bsz, hp, wpd, n), _BF)
        out_spec = pl.BlockSpec((bc, hp, wpd, n), lambda i: (i, 0, 0, 0))
    body = functools.partial(_conv3s1_body, bc=bc, ho=ho, wo=wo, cin=cin, n=n,
                             res=res, head=head)
    return pl.pallas_call(
        body,
        out_shape=out_shape,
        grid=(bsz // bc,),
        in_specs=specs,
        out_specs=out_spec,
        compiler_params=pltpu.CompilerParams(dimension_semantics=("parallel",)),
    )(*ins)


def _phase_split(xp):
    return tuple(xp[:, a::2, b::2, :] for a in range(2) for b in range(2))


def _chunk(bsz, want):
    c = min(want, bsz)
    while bsz % c:
        c -= 1
    return c


def kernel(x, conv1_wmat, conv1_scale, conv1_bias, l0b0_c1_wmat, l0b0_c1_scale, l0b0_c1_bias, l0b0_c2_wmat, l0b0_c2_scale, l0b0_c2_bias, l0b1_c1_wmat, l0b1_c1_scale, l0b1_c1_bias, l0b1_c2_wmat, l0b1_c2_scale, l0b1_c2_bias, l1b0_c1_wmat, l1b0_c1_scale, l1b0_c1_bias, l1b0_c2_wmat, l1b0_c2_scale, l1b0_c2_bias, l1b0_ds_wmat, l1b0_ds_scale, l1b0_ds_bias, l1b1_c1_wmat, l1b1_c1_scale, l1b1_c1_bias, l1b1_c2_wmat, l1b1_c2_scale, l1b1_c2_bias, l2b0_c1_wmat, l2b0_c1_scale, l2b0_c1_bias, l2b0_c2_wmat, l2b0_c2_scale, l2b0_c2_bias, l2b0_ds_wmat, l2b0_ds_scale, l2b0_ds_bias, l2b1_c1_wmat, l2b1_c1_scale, l2b1_c1_bias, l2b1_c2_wmat, l2b1_c2_scale, l2b1_c2_bias, l3b0_c1_wmat, l3b0_c1_scale, l3b0_c1_bias, l3b0_c2_wmat, l3b0_c2_scale, l3b0_c2_bias, l3b0_ds_wmat, l3b0_ds_scale, l3b0_ds_bias, l3b1_c1_wmat, l3b1_c1_scale, l3b1_c1_bias, l3b1_c2_wmat, l3b1_c2_scale, l3b1_c2_bias, fc_w, fc_b):
    x = x.reshape(-1, 3, 224, 224)
    bsz = x.shape[0]
    bc2 = _chunk(bsz, 4)
    bc3 = _chunk(bsz, 16)
    bc4 = _chunk(bsz, 32)

    po = _stem_pool(_stem_s2d(x), _stem_w16(conv1_wmat), conv1_scale,
                    conv1_bias, _stem_mask())
    p1c = po.reshape(bsz, 56, 120, 64)[:, :, 2:114:2, :]
    p1 = jnp.pad(p1c, ((0, 0), (1, 1), (1, 7), (0, 0))).reshape(
        bsz, 58 * 64, 64)

    mk1 = _flat_mask(58, 64, 56, 56)
    y = _fs1(p1, l0b0_c1_wmat, l0b0_c1_scale, l0b0_c1_bias, mk1, 64)
    p2 = _fs1(y, l0b0_c2_wmat, l0b0_c2_scale, l0b0_c2_bias, mk1, 64, r=p1)
    y = _fs1(p2, l0b1_c1_wmat, l0b1_c1_scale, l0b1_c1_bias, mk1, 64)
    p3 = _fs1(y, l0b1_c2_wmat, l0b1_c2_scale, l0b1_c2_bias, mk1, 64, r=p2)

    mk2 = _flat_mask(30, 32, 28, 28)
    y1, idn = _conv3s2_ds(_phase_split(p3.reshape(bsz, 58, 64, 64)),
                          l1b0_c1_wmat, l1b0_c1_scale, l1b0_c1_bias,
                          l1b0_ds_wmat, l1b0_ds_scale, l1b0_ds_bias,
                          bc=bc2, ho=28, wo=28, wpad=32)
    y1 = y1.reshape(bsz, 960, 128)
    idn = idn.reshape(bsz, 960, 128)
    p4 = _fs1(y1, l1b0_c2_wmat, l1b0_c2_scale, l1b0_c2_bias, mk2, 32, r=idn)
    y = _fs1(p4, l1b1_c1_wmat, l1b1_c1_scale, l1b1_c1_bias, mk2, 32)
    p5 = _fs1(y, l1b1_c2_wmat, l1b1_c2_scale, l1b1_c2_bias, mk2, 32, r=p4)

    mk3 = _flat_mask(16, 16, 14, 14)
    y1, idn = _conv3s2_ds(_phase_split(p5.reshape(bsz, 30, 32, 128)),
                          l2b0_c1_wmat, l2b0_c1_scale, l2b0_c1_bias,
                          l2b0_ds_wmat, l2b0_ds_scale, l2b0_ds_bias,
                          bc=bc3, ho=14, wo=14, wpad=16)
    y1 = y1.reshape(bsz, 256, 256)
    idn = idn.reshape(bsz, 256, 256)
    p6 = _fs1(y1, l2b0_c2_wmat, l2b0_c2_scale, l2b0_c2_bias, mk3, 16, r=idn)
    y = _fs1(p6, l2b1_c1_wmat, l2b1_c1_scale, l2b1_c1_bias, mk3, 16)
    p7 = _fs1(y, l2b1_c2_wmat, l2b1_c2_scale, l2b1_c2_bias, mk3, 16, r=p6)

    y1p, idn = _conv3s2_ds(_phase_split(p7.reshape(bsz, 16, 16, 256)),
                           l3b0_c1_wmat, l3b0_c1_scale, l3b0_c1_bias,
                           l3b0_ds_wmat, l3b0_ds_scale, l3b0_ds_bias,
                           bc=bc4, ho=7, wo=7, wpad=9)
    p8 = _conv3s1(y1p, l3b0_c2_wmat, l3b0_c2_scale, l3b0_c2_bias, bc=bc4,
                  res="padded", r=idn)
    y = _conv3s1(p8, l3b1_c1_wmat, l3b1_c1_scale, l3b1_c1_bias, bc=bc4)
    return _conv3s1(y, l3b1_c2_wmat, l3b1_c2_scale, l3b1_c2_bias, bc=bc4,
                    res="padded", r=p8, head=True, fcw=fc_w, fcb=fc_b)
